# Initial kernel scaffold; baseline (speedup 1.0000x reference)
import jax
import jax.numpy as jnp
from jax.experimental import pallas as pl


def kernel(x, edge_index, Wl1, bl1, Wr1, Wl2, bl2, Wr2, Wl3, bl3, Wr3):
    raise NotImplementedError("write your pallas kernel here")


# speedup vs baseline: 6.1336x; 6.1336x over previous
import functools

import jax
import jax.numpy as jnp
from jax import lax
from jax.experimental import pallas as pl
from jax.experimental.pallas import tpu as pltpu
from jax.experimental.pallas import tpu_sc as plsc

N = 10000
E = 320000
D = 128

NC = 2
NS = 16
NW = NC * NS
EPW = E // NW
CH = 80
NCH = EPW // CH
GC = 25
NG = NCH // GC
NP = 10240
RPT = NP // NS

_mesh = plsc.VectorSubcoreMesh(core_axis_name="c", subcore_axis_name="s")


def _fill_rows(rows_v, val16):
    def _row(r, _):
        for j in range(D // 16):
            rows_v[r, pl.ds(j * 16, 16)] = val16
        return 0

    lax.fori_loop(0, CH, _row, 0)


def _agg_body(with_gather, *refs):
    if with_gather:
        (h_hbm, src_hbm, dst_hbm, out_hbm, acc, src_v, dst_v, rows_v, sem) = refs
    else:
        (dst_hbm, out_hbm, acc, dst_v, rows_v, sem) = refs

    cid = lax.axis_index("c")
    sid = lax.axis_index("s")
    wid = sid * NC + cid

    _fill_rows(rows_v, jnp.zeros((16,), jnp.float32))
    for k in range(RPT // CH):
        pltpu.sync_copy(rows_v, acc.at[pl.ds(sid * RPT + k * CH, CH)])

    if not with_gather:
        _fill_rows(rows_v, jnp.ones((16,), jnp.float32))

    plsc.subcore_barrier()

    def _step(c, _):
        if with_gather:
            pltpu.async_copy(h_hbm.at[src_v.at[c]], rows_v, sem).wait()
        pltpu.sync_copy(rows_v, acc.at[dst_v.at[c]], add=True)
        return 0

    for g in range(NG):
        if with_gather:
            pltpu.sync_copy(src_hbm.at[wid, g], src_v)
        pltpu.sync_copy(dst_hbm.at[wid, g], dst_v)
        lax.fori_loop(0, GC, _step, 0)

    plsc.subcore_barrier()

    for k in range(RPT // CH):
        base = sid * RPT + k * CH
        pltpu.sync_copy(acc.at[pl.ds(base, CH)], rows_v)
        pltpu.sync_copy(rows_v, out_hbm.at[cid, pl.ds(base, CH)])


def _make_agg(with_gather):
    scratch = [
        pltpu.VMEM_SHARED((NP, D), jnp.float32),
    ]
    if with_gather:
        scratch.append(pltpu.VMEM((GC, CH), jnp.int32))
    scratch += [
        pltpu.VMEM((GC, CH), jnp.int32),
        pltpu.VMEM((CH, D), jnp.float32),
        pltpu.SemaphoreType.DMA,
    ]
    return pl.kernel(
        functools.partial(_agg_body, with_gather),
        out_type=jax.ShapeDtypeStruct((NC, NP, D), jnp.float32),
        mesh=_mesh,
        scratch_types=scratch,
    )


_agg = _make_agg(True)
_deg = _make_agg(False)

_RB = 400
_GRID = N // _RB


def _combine_body(relu, p_ref, dg_ref, x_ref, wl_ref, bl_ref, wr_ref, o_ref):
    p = p_ref[0] + p_ref[1]
    deg = dg_ref[0] + dg_ref[1]
    mean = p / jnp.maximum(deg, 1.0)
    acc = jnp.dot(mean, wl_ref[...], preferred_element_type=jnp.float32)
    acc = acc + jnp.dot(x_ref[...], wr_ref[...], preferred_element_type=jnp.float32)
    acc = acc + bl_ref[...]
    o_ref[...] = jnp.maximum(acc, 0.0) if relu else acc


def _make_combine(relu):
    return pl.pallas_call(
        functools.partial(_combine_body, relu),
        grid=(_GRID,),
        in_specs=[
            pl.BlockSpec((NC, _RB, D), lambda i: (0, i, 0)),
            pl.BlockSpec((NC, _RB, D), lambda i: (0, i, 0)),
            pl.BlockSpec((_RB, D), lambda i: (i, 0)),
            pl.BlockSpec((D, D), lambda i: (0, 0)),
            pl.BlockSpec((1, D), lambda i: (0, 0)),
            pl.BlockSpec((D, D), lambda i: (0, 0)),
        ],
        out_specs=pl.BlockSpec((_RB, D), lambda i: (i, 0)),
        out_shape=jax.ShapeDtypeStruct((N, D), jnp.float32),
    )


_combine_relu = _make_combine(True)
_combine_lin = _make_combine(False)


def kernel(x, edge_index, Wl1, bl1, Wr1, Wl2, bl2, Wr2, Wl3, bl3, Wr3):
    src = edge_index[0].reshape(NW, NG, GC, CH)
    dst = edge_index[1].reshape(NW, NG, GC, CH)
    bl1r = bl1.reshape(1, D)
    bl2r = bl2.reshape(1, D)
    bl3r = bl3.reshape(1, D)

    degp = _deg(dst)
    agg1 = _agg(x, src, dst)
    h = _combine_relu(agg1, degp, x, Wl1, bl1r, Wr1)
    agg2 = _agg(h, src, dst)
    h_out = _combine_relu(agg2, degp, h, Wl2, bl2r, Wr2)
    agg3 = _agg(h_out, src, dst)
    out = _combine_lin(agg3, degp, h_out, Wl3, bl3r, Wr3)
    return (out, h_out)


# speedup vs baseline: 8.8970x; 1.4505x over previous
import functools

import jax
import jax.numpy as jnp
from jax import lax
from jax.experimental import pallas as pl
from jax.experimental.pallas import tpu as pltpu
from jax.experimental.pallas import tpu_sc as plsc

N = 10000
E = 320000
D = 128

NC = 2
NS = 16
NW = NC * NS
EPW = E // NW
CH = 80
NCH = EPW // CH
GC = 25
NG = NCH // GC
NP = 10240
RPT = NP // NS

_mesh = plsc.VectorSubcoreMesh(core_axis_name="c", subcore_axis_name="s")


def _fill_rows(rows_v, val16):
    def _row(r, _):
        for j in range(D // 16):
            rows_v[r, pl.ds(j * 16, 16)] = val16
        return 0

    lax.fori_loop(0, CH, _row, 0)


def _agg_body(with_gather, *refs):
    if with_gather:
        (h_hbm, src_hbm, dst_hbm, out_hbm,
         acc, src_v, dst_v, rows_v, sem_a, sem_b) = refs
        buf_a = rows_v.at[0]
        buf_b = rows_v.at[1]
    else:
        (dst_hbm, out_hbm, acc, dst_v, rows_v, sem_a) = refs
        buf_a = rows_v

    cid = lax.axis_index("c")
    sid = lax.axis_index("s")
    wid = sid * NC + cid

    _fill_rows(buf_a, jnp.zeros((16,), jnp.float32))
    for k in range(RPT // CH):
        pltpu.sync_copy(buf_a, acc.at[pl.ds(sid * RPT + k * CH, CH)])

    if not with_gather:
        _fill_rows(buf_a, jnp.ones((16,), jnp.float32))

    plsc.subcore_barrier()

    if with_gather:
        bufs = (buf_a, buf_b)
        sems = (sem_a, sem_b)

        def _group(g, _):
            pltpu.sync_copy(src_hbm.at[wid, g], src_v)
            pltpu.sync_copy(dst_hbm.at[wid, g], dst_v)
            pends = [pltpu.async_copy(h_hbm.at[src_v.at[0]], bufs[0], sems[0]),
                     None]
            for c in range(GC):
                b = c % 2
                if c + 1 < GC:
                    nb = (c + 1) % 2
                    pends[nb] = pltpu.async_copy(
                        h_hbm.at[src_v.at[c + 1]], bufs[nb], sems[nb])
                pends[b].wait()
                pltpu.sync_copy(bufs[b], acc.at[dst_v.at[c]], add=True)
            return 0

        lax.fori_loop(0, NG, _group, 0)
    else:
        def _step(c, _):
            pltpu.sync_copy(buf_a, acc.at[dst_v.at[c]], add=True)
            return 0

        for g in range(NG):
            pltpu.sync_copy(dst_hbm.at[wid, g], dst_v)
            lax.fori_loop(0, GC, _step, 0)

    plsc.subcore_barrier()

    for k in range(RPT // CH):
        base = sid * RPT + k * CH
        pltpu.sync_copy(acc.at[pl.ds(base, CH)], buf_a)
        pltpu.sync_copy(buf_a, out_hbm.at[cid, pl.ds(base, CH)])


def _make_agg(with_gather):
    scratch = [
        pltpu.VMEM_SHARED((NP, D), jnp.float32),
    ]
    if with_gather:
        scratch += [
            pltpu.VMEM((GC, CH), jnp.int32),
            pltpu.VMEM((GC, CH), jnp.int32),
            pltpu.VMEM((2, CH, D), jnp.float32),
            pltpu.SemaphoreType.DMA,
            pltpu.SemaphoreType.DMA,
        ]
    else:
        scratch += [
            pltpu.VMEM((GC, CH), jnp.int32),
            pltpu.VMEM((CH, D), jnp.float32),
            pltpu.SemaphoreType.DMA,
        ]
    return pl.kernel(
        functools.partial(_agg_body, with_gather),
        out_type=jax.ShapeDtypeStruct((NC, NP, D), jnp.float32),
        mesh=_mesh,
        scratch_types=scratch,
    )


_agg = _make_agg(True)
_deg = _make_agg(False)

_RB = 400
_GRID = N // _RB


def _combine_body(relu, p_ref, dg_ref, x_ref, wl_ref, bl_ref, wr_ref, o_ref):
    p = p_ref[0] + p_ref[1]
    deg = dg_ref[0] + dg_ref[1]
    mean = p / jnp.maximum(deg, 1.0)
    acc = jnp.dot(mean, wl_ref[...], preferred_element_type=jnp.float32)
    acc = acc + jnp.dot(x_ref[...], wr_ref[...], preferred_element_type=jnp.float32)
    acc = acc + bl_ref[...]
    o_ref[...] = jnp.maximum(acc, 0.0) if relu else acc


def _make_combine(relu):
    return pl.pallas_call(
        functools.partial(_combine_body, relu),
        grid=(_GRID,),
        in_specs=[
            pl.BlockSpec((NC, _RB, D), lambda i: (0, i, 0)),
            pl.BlockSpec((NC, _RB, D), lambda i: (0, i, 0)),
            pl.BlockSpec((_RB, D), lambda i: (i, 0)),
            pl.BlockSpec((D, D), lambda i: (0, 0)),
            pl.BlockSpec((1, D), lambda i: (0, 0)),
            pl.BlockSpec((D, D), lambda i: (0, 0)),
        ],
        out_specs=pl.BlockSpec((_RB, D), lambda i: (i, 0)),
        out_shape=jax.ShapeDtypeStruct((N, D), jnp.float32),
    )


_combine_relu = _make_combine(True)
_combine_lin = _make_combine(False)


def kernel(x, edge_index, Wl1, bl1, Wr1, Wl2, bl2, Wr2, Wl3, bl3, Wr3):
    src = edge_index[0].reshape(NW, NG, GC, CH)
    dst = edge_index[1].reshape(NW, NG, GC, CH)
    bl1r = bl1.reshape(1, D)
    bl2r = bl2.reshape(1, D)
    bl3r = bl3.reshape(1, D)

    degp = _deg(dst)
    agg1 = _agg(x, src, dst)
    h = _combine_relu(agg1, degp, x, Wl1, bl1r, Wr1)
    agg2 = _agg(h, src, dst)
    h_out = _combine_relu(agg2, degp, h, Wl2, bl2r, Wr2)
    agg3 = _agg(h_out, src, dst)
    out = _combine_lin(agg3, degp, h_out, Wl3, bl3r, Wr3)
    return (out, h_out)


# speedup vs baseline: 10.0324x; 1.1276x over previous
import functools

import jax
import jax.numpy as jnp
from jax import lax
from jax.experimental import pallas as pl
from jax.experimental.pallas import tpu as pltpu
from jax.experimental.pallas import tpu_sc as plsc

N = 10000
E = 320000
D = 128

NC = 2
NS = 16
NW = NC * NS
EPW = E // NW
CH = 80
NCH = EPW // CH
GC = 25
NG = NCH // GC
NP = 10240
RPT = NP // NS

_mesh = plsc.VectorSubcoreMesh(core_axis_name="c", subcore_axis_name="s")


def _fill_rows(rows_v, val16):
    def _row(r, _):
        for j in range(D // 16):
            rows_v[r, pl.ds(j * 16, 16)] = val16
        return 0

    lax.fori_loop(0, CH, _row, 0)


NB = 3


def _agg_body(with_gather, *refs):
    if with_gather:
        (h_hbm, src_hbm, dst_hbm, out_hbm,
         acc, src_v, dst_v, rows_v, *sems) = refs
        bufs = [rows_v.at[i] for i in range(NB)]
        buf_a = bufs[0]
        buf_b = bufs[1]
    else:
        (dst_hbm, out_hbm, acc, dst_v, rows_v, *sems) = refs
        buf_a = rows_v

    cid = lax.axis_index("c")
    sid = lax.axis_index("s")
    wid = sid * NC + cid

    _fill_rows(buf_a, jnp.zeros((16,), jnp.float32))
    for k in range(RPT // CH):
        pltpu.sync_copy(buf_a, acc.at[pl.ds(sid * RPT + k * CH, CH)])

    if not with_gather:
        _fill_rows(buf_a, jnp.ones((16,), jnp.float32))

    plsc.subcore_barrier()

    if with_gather:
        gsems = sems[:NB]
        ssems = sems[NB:]

        def _group(g, _):
            pltpu.sync_copy(src_hbm.at[wid, g], src_v)
            pltpu.sync_copy(dst_hbm.at[wid, g], dst_v)
            pend_g = [None] * NB
            pend_s = [None] * NB
            pend_g[0] = pltpu.async_copy(h_hbm.at[src_v.at[0]], bufs[0],
                                         gsems[0])
            for c in range(GC):
                b = c % NB
                if c + 1 < GC:
                    nb = (c + 1) % NB
                    if pend_s[nb] is not None:
                        pend_s[nb].wait()
                        pend_s[nb] = None
                    pend_g[nb] = pltpu.async_copy(
                        h_hbm.at[src_v.at[c + 1]], bufs[nb], gsems[nb])
                pend_g[b].wait()
                pend_s[b] = pltpu.make_async_copy(
                    bufs[b], acc.at[dst_v.at[c]], ssems[b])
                pend_s[b].start(add=True)
            for b in range(NB):
                if pend_s[b] is not None:
                    pend_s[b].wait()
            return 0

        lax.fori_loop(0, NG, _group, 0)
    else:
        ssems = sems

        def _group(g, _):
            pltpu.sync_copy(dst_hbm.at[wid, g], dst_v)
            pend_s = [None] * NB
            for c in range(GC):
                b = c % NB
                if pend_s[b] is not None:
                    pend_s[b].wait()
                pend_s[b] = pltpu.make_async_copy(
                    buf_a, acc.at[dst_v.at[c]], ssems[b])
                pend_s[b].start(add=True)
            for b in range(NB):
                if pend_s[b] is not None:
                    pend_s[b].wait()
            return 0

        lax.fori_loop(0, NG, _group, 0)

    plsc.subcore_barrier()

    dbufs = (buf_a, buf_b) if with_gather else (buf_a, buf_a)
    dsems = (sems[0], sems[1]) if with_gather else (sems[0], sems[0])
    NK = RPT // CH
    pend = [None, None]
    pend[0] = pltpu.async_copy(acc.at[pl.ds(sid * RPT, CH)], dbufs[0],
                               dsems[0])
    for k in range(NK):
        b = k % 2 if with_gather else 0
        pend[b].wait()
        if with_gather and k + 1 < NK:
            nb = (k + 1) % 2
            pend[nb] = pltpu.async_copy(
                acc.at[pl.ds(sid * RPT + (k + 1) * CH, CH)], dbufs[nb],
                dsems[nb])
        pltpu.sync_copy(dbufs[b], out_hbm.at[cid, pl.ds(sid * RPT + k * CH, CH)])
        if not with_gather and k + 1 < NK:
            pend[0] = pltpu.async_copy(
                acc.at[pl.ds(sid * RPT + (k + 1) * CH, CH)], dbufs[0],
                dsems[0])


def _make_agg(with_gather):
    scratch = [
        pltpu.VMEM_SHARED((NP, D), jnp.float32),
    ]
    if with_gather:
        scratch += [
            pltpu.VMEM((GC, CH), jnp.int32),
            pltpu.VMEM((GC, CH), jnp.int32),
            pltpu.VMEM((NB, CH, D), jnp.float32),
        ] + [pltpu.SemaphoreType.DMA] * (2 * NB)
    else:
        scratch += [
            pltpu.VMEM((GC, CH), jnp.int32),
            pltpu.VMEM((CH, D), jnp.float32),
        ] + [pltpu.SemaphoreType.DMA] * NB
    return pl.kernel(
        functools.partial(_agg_body, with_gather),
        out_type=jax.ShapeDtypeStruct((NC, NP, D), jnp.float32),
        mesh=_mesh,
        scratch_types=scratch,
    )


_agg = _make_agg(True)
_deg = _make_agg(False)

_RB = 400
_GRID = N // _RB


def _combine_body(relu, p_ref, dg_ref, x_ref, wl_ref, bl_ref, wr_ref, o_ref):
    p = p_ref[0] + p_ref[1]
    deg = dg_ref[0] + dg_ref[1]
    mean = p / jnp.maximum(deg, 1.0)
    acc = jnp.dot(mean, wl_ref[...], preferred_element_type=jnp.float32)
    acc = acc + jnp.dot(x_ref[...], wr_ref[...], preferred_element_type=jnp.float32)
    acc = acc + bl_ref[...]
    o_ref[...] = jnp.maximum(acc, 0.0) if relu else acc


def _make_combine(relu):
    return pl.pallas_call(
        functools.partial(_combine_body, relu),
        grid=(_GRID,),
        in_specs=[
            pl.BlockSpec((NC, _RB, D), lambda i: (0, i, 0)),
            pl.BlockSpec((NC, _RB, D), lambda i: (0, i, 0)),
            pl.BlockSpec((_RB, D), lambda i: (i, 0)),
            pl.BlockSpec((D, D), lambda i: (0, 0)),
            pl.BlockSpec((1, D), lambda i: (0, 0)),
            pl.BlockSpec((D, D), lambda i: (0, 0)),
        ],
        out_specs=pl.BlockSpec((_RB, D), lambda i: (i, 0)),
        out_shape=jax.ShapeDtypeStruct((N, D), jnp.float32),
    )


_combine_relu = _make_combine(True)
_combine_lin = _make_combine(False)


def kernel(x, edge_index, Wl1, bl1, Wr1, Wl2, bl2, Wr2, Wl3, bl3, Wr3):
    src = edge_index[0].reshape(NW, NG, GC, CH)
    dst = edge_index[1].reshape(NW, NG, GC, CH)
    bl1r = bl1.reshape(1, D)
    bl2r = bl2.reshape(1, D)
    bl3r = bl3.reshape(1, D)

    degp = _deg(dst)
    agg1 = _agg(x, src, dst)
    h = _combine_relu(agg1, degp, x, Wl1, bl1r, Wr1)
    agg2 = _agg(h, src, dst)
    h_out = _combine_relu(agg2, degp, h, Wl2, bl2r, Wr2)
    agg3 = _agg(h_out, src, dst)
    out = _combine_lin(agg3, degp, h_out, Wl3, bl3r, Wr3)
    return (out, h_out)
